# Initial kernel scaffold; baseline (speedup 1.0000x reference)
import jax
import jax.numpy as jnp
from jax.experimental import pallas as pl


def kernel(coords, atom_types, t, batch, time_W, edge_W, params):
    raise NotImplementedError("write your pallas kernel here")


# speedup vs baseline: 8.5149x; 8.5149x over previous
import jax
import jax.numpy as jnp
import numpy as np
from jax.experimental import pallas as pl
from jax.experimental.pallas import tpu as pltpu

N = 128
H = 256
L = 6
S = 100
ED = 64
CUTOFF = 8.0
R = 64
T = N // R
RN = R * N


def _dot(a, b):
    return jax.lax.dot_general(
        a, b, (((a.ndim - 1,), (0,)), ((), ())),
        precision=jax.lax.Precision.HIGHEST,
        preferred_element_type=jnp.float32)


def _dotd(a, b):
    return jax.lax.dot_general(
        a, b, (((a.ndim - 1,), (0,)), ((), ())),
        preferred_element_type=jnp.float32)


def _rep_rows(x, reps):
    a, f = x.shape
    return jnp.broadcast_to(x[:, None, :], (a, reps, f)).reshape(a * reps, f)


def _tile_rows(x, reps):
    a, f = x.shape
    return jnp.broadcast_to(x[None, :, :], (reps, a, f)).reshape(reps * a, f)


def _body(c0, ids, t11, timeW, edgeW, atab,
          tw1, tb1, tw2, tb2,
          ew1, eb1, ew2e, eb2e,
          Wa, Wb, Wcb, ewb, ew2l, ew2b,
          cwl, cwb, cw2l, cw2b,
          nwl, nwb, nw2l, nw2b,
          cp1, cpb1, cp2, cpb2,
          tp1, tpb1, tp2, tpb2,
          out_cn, out_tl,
          ea_s, mask_s, nf_s, a_s, bt_s, nmsg_s, cupd_s, c_s):
    silu = jax.nn.silu
    l = pl.program_id(0)
    t = pl.program_id(1)
    r0 = t * R

    @pl.when(jnp.logical_and(l == 0, t == 0))
    def _init():
        xp = t11[...] * timeW[...] * (2.0 * np.pi)
        gf = jnp.concatenate([jnp.sin(xp), jnp.cos(xp)], axis=1)
        te = _dotd(silu(_dotd(gf, tw1[...]) + tb1[...]), tw2[...]) + tb2[...]
        oh = (jax.lax.broadcasted_iota(jnp.int32, (N, S), 1)
              == ids[...]).astype(jnp.float32)
        nf_s[...] = _dot(oh, atab[...]) + te
        c_s[...] = c0[...]

    @pl.when(t == 0)
    def _layer_pre():
        nf = nf_s[...]
        a_s[...] = _dotd(nf, Wa[0])
        bt_s[...] = _tile_rows(_dotd(nf, Wb[0]), R) + ewb[0]

    @pl.when(l == 0)
    def _edge_embed():
        c0r = c0[pl.ds(r0, R), :]
        d0c = _rep_rows(c0r, N) - _tile_rows(c0[...], R)
        d0 = jnp.sqrt(jnp.sum(d0c * d0c, axis=1, keepdims=True))
        k = jax.lax.broadcasted_iota(jnp.int32, (RN, 1), 0)
        gi = jax.lax.shift_right_logical(k, 7) + r0
        gj = jnp.bitwise_and(k, N - 1)
        mask_s[pl.ds(r0 * N, RN), :] = jnp.logical_and(
            d0 < CUTOFF, gi != gj).astype(jnp.float32)
        xp = d0 * edgeW[...] * (2.0 * np.pi)
        gf = jnp.concatenate([jnp.sin(xp), jnp.cos(xp)], axis=1)
        ea_s[pl.ds(r0 * N, RN), :] = (
            _dotd(silu(_dotd(gf, ew1[...]) + eb1[...]), ew2e[...])
            + eb2e[...]).astype(jnp.bfloat16)

    mask = mask_s[pl.ds(r0 * N, RN), :]

    ea_t = ea_s[pl.ds(r0 * N, RN), :]
    at = a_s[pl.ds(r0, R), :]
    h1 = _dotd(ea_t, Wcb[0]) + _rep_rows(at, N) + bt_s[...]
    em = _dotd(silu(h1), ew2l[0]) + ew2b[0]
    g = silu(_dotd(em, cwl[0]) + cwb[0])
    cg = _dotd(g, cw2l[0]) + cw2b[0]

    cr = c_s[pl.ds(r0, R), :]
    cd = _rep_rows(cr, N) - _tile_rows(c_s[...], R)
    cdist = jnp.sqrt(jnp.sum(cd * cd, axis=1, keepdims=True)) + 1e-8
    cm = cg * cd / cdist * mask

    cupd_s[pl.ds(r0, R), :] = jnp.sum(cm.reshape(R, N, 3), axis=1)
    nmsg_s[pl.ds(r0, R), :] = jnp.sum((em * mask).reshape(R, N, H), axis=1)

    @pl.when(t == T - 1)
    def _layer_post():
        nf = nf_s[...]
        nm = nmsg_s[...]
        hh = silu(_dotd(nf, nwl[0, :H, :]) + _dotd(nm, nwl[0, H:, :]) + nwb[0])
        nf_s[...] = _dotd(hh, nw2l[0]) + nw2b[0]
        c_s[...] = c_s[...] + cupd_s[...]

    @pl.when(jnp.logical_and(l == L - 1, t == T - 1))
    def _heads():
        nf = nf_s[...]
        out_cn[...] = _dotd(silu(_dotd(nf, cp1[...]) + cpb1[...]), cp2[...]) + cpb2[...]
        out_tl[...] = _dotd(silu(_dotd(nf, tp1[...]) + tpb1[...]), tp2[...]) + tpb2[...]


def _full(shape):
    nd = len(shape)
    return pl.BlockSpec(shape, lambda l, t, _n=nd: (0,) * _n)


def _per_layer(shape):
    nd = len(shape)
    return pl.BlockSpec((1,) + shape, lambda l, t, _n=nd: (l,) + (0,) * _n)


def kernel(coords, atom_types, t, batch, time_W, edge_W, params):
    p = params
    ids = atom_types.reshape(N, 1).astype(jnp.int32)
    t11 = t.astype(jnp.float32).reshape(1, 1)
    timeW = time_W.reshape(1, H // 2)
    edgeW = edge_W.reshape(1, ED // 2)
    Wa = p['ew'][:, :H, :]
    Wb = p['ew'][:, H:2 * H, :]
    Wcb = p['ew'][:, 2 * H:, :].astype(jnp.bfloat16)

    args = (
        coords, ids, t11, timeW, edgeW, p['atom_table'],
        p['time_w1'], p['time_b1'].reshape(1, -1),
        p['time_w2'], p['time_b2'].reshape(1, -1),
        p['edge_w1'], p['edge_b1'].reshape(1, -1),
        p['edge_w2'], p['edge_b2'].reshape(1, -1),
        Wa, Wb, Wcb, p['ew_b'].reshape(L, 1, H),
        p['ew2'], p['ew2_b'].reshape(L, 1, H),
        p['cw'], p['cw_b'].reshape(L, 1, H),
        p['cw2'], p['cw2_b'].reshape(L, 1, 1),
        p['nw'], p['nw_b'].reshape(L, 1, H),
        p['nw2'], p['nw2_b'].reshape(L, 1, H),
        p['cp_w1'], p['cp_b1'].reshape(1, H),
        p['cp_w2'], p['cp_b2'].reshape(1, 3),
        p['tp_w1'], p['tp_b1'].reshape(1, H),
        p['tp_w2'], p['tp_b2'].reshape(1, S),
    )
    in_specs = [
        _full((N, 3)), _full((N, 1)), _full((1, 1)),
        _full((1, H // 2)), _full((1, ED // 2)), _full((S, H)),
        _full((H, 4 * H)), _full((1, 4 * H)),
        _full((4 * H, H)), _full((1, H)),
        _full((ED, ED)), _full((1, ED)),
        _full((ED, ED)), _full((1, ED)),
        _per_layer((H, H)), _per_layer((H, H)), _per_layer((ED, H)),
        _per_layer((1, H)),
        _per_layer((H, H)), _per_layer((1, H)),
        _per_layer((H, H)), _per_layer((1, H)),
        _per_layer((H, 1)), _per_layer((1, 1)),
        _per_layer((2 * H, H)), _per_layer((1, H)),
        _per_layer((H, H)), _per_layer((1, H)),
        _full((H, H)), _full((1, H)), _full((H, 3)), _full((1, 3)),
        _full((H, H)), _full((1, H)), _full((H, S)), _full((1, S)),
    ]
    out_shape = (
        jax.ShapeDtypeStruct((N, 3), jnp.float32),
        jax.ShapeDtypeStruct((N, S), jnp.float32),
    )
    out_specs = (_full((N, 3)), _full((N, S)))
    scratch_shapes = [
        pltpu.VMEM((N * N, ED), jnp.bfloat16),
        pltpu.VMEM((N * N, 1), jnp.float32),
        pltpu.VMEM((N, H), jnp.float32),
        pltpu.VMEM((N, H), jnp.float32),
        pltpu.VMEM((RN, H), jnp.float32),
        pltpu.VMEM((N, H), jnp.float32),
        pltpu.VMEM((N, 3), jnp.float32),
        pltpu.VMEM((N, 3), jnp.float32),
    ]
    cn, tl = pl.pallas_call(
        _body,
        grid=(L, T),
        in_specs=in_specs,
        out_specs=out_specs,
        out_shape=out_shape,
        scratch_shapes=scratch_shapes,
        compiler_params=pltpu.CompilerParams(
            dimension_semantics=("arbitrary", "arbitrary")),
    )(*args)
    return (cn, tl)
